# baseline (speedup 1.0000x reference)
import functools

import jax
import jax.numpy as jnp
from jax import lax
from jax.experimental import pallas as pl
from jax.experimental.pallas import tpu as pltpu
from jax.experimental.pallas import tpu_sc as plsc


def _make_gather(B: int, D: int, vocab: int):
    info = plsc.get_sparse_core_info()
    nw = info.num_cores * info.num_subcores
    assert B % (8 * nw) == 0
    b_per_w = B // nw
    ch = 8
    nbuf = 12
    n_ch = b_per_w // ch
    mesh = plsc.VectorSubcoreMesh(core_axis_name="c", subcore_axis_name="s")

    @functools.partial(
        pl.kernel,
        mesh=mesh,
        out_type=jax.ShapeDtypeStruct((B, D), jnp.float32),
        scratch_types=[
            pltpu.VMEM((b_per_w,), jnp.int32),
            pltpu.VMEM((nbuf, ch, D), jnp.float32),
        ] + [pltpu.SemaphoreType.DMA] * (2 * nbuf),
    )
    def gather(table_hbm, idx_hbm, out_hbm, idx_v, rows_v, *sems):
        gsem, wsem = sems[:nbuf], sems[nbuf:]
        wid = lax.axis_index("s") * info.num_cores + lax.axis_index("c")
        base = wid * b_per_w
        pltpu.sync_copy(idx_hbm.at[pl.ds(base, b_per_w)], idx_v)
        for t in range(b_per_w // 16):
            sl = pl.ds(t * 16, 16)
            idx_v[sl] = jnp.clip(idx_v[sl], 0, vocab - 1)

        def start_gather(i):
            return pltpu.async_copy(
                table_hbm.at[idx_v.at[pl.ds(i * ch, ch)]],
                rows_v.at[i % nbuf], gsem[i % nbuf])

        def start_write(i):
            return pltpu.async_copy(
                rows_v.at[i % nbuf], out_hbm.at[pl.ds(base + i * ch, ch)],
                wsem[i % nbuf])

        gathers = [None] * n_ch
        writes = [None] * n_ch
        for j in range(min(nbuf - 1, n_ch)):
            gathers[j] = start_gather(j)
        for i in range(n_ch):
            j = i + nbuf - 1
            if j < n_ch:
                if j - nbuf >= 0:
                    writes[j - nbuf].wait()
                gathers[j] = start_gather(j)
            gathers[i].wait()
            writes[i] = start_write(i)
        for i in range(max(0, n_ch - nbuf), n_ch):
            writes[i].wait()

    return gather


def kernel(ids, ids_valid, ids_mask, embed_table):
    vocab, d = embed_table.shape
    b, s = ids.shape
    ids_flat = ids.reshape(-1)
    out = _make_gather(b * s, d, vocab)(embed_table, ids_flat)
    return (out.reshape(b, s, d), ids_valid, ids, ids_mask)


# speedup vs baseline: 1.0185x; 1.0185x over previous
import functools

import jax
import jax.numpy as jnp
from jax import lax
from jax.experimental import pallas as pl
from jax.experimental.pallas import tpu as pltpu
from jax.experimental.pallas import tpu_sc as plsc


def _make_gather(B: int, D: int, vocab: int):
    info = plsc.get_sparse_core_info()
    nw = info.num_cores * info.num_subcores
    assert B % (8 * nw) == 0
    b_per_w = B // nw
    ch = 16
    nbuf = 7
    n_ch = b_per_w // ch
    mesh = plsc.VectorSubcoreMesh(core_axis_name="c", subcore_axis_name="s")

    @functools.partial(
        pl.kernel,
        mesh=mesh,
        out_type=jax.ShapeDtypeStruct((B, D), jnp.float32),
        scratch_types=[
            pltpu.VMEM((b_per_w,), jnp.int32),
            pltpu.VMEM((nbuf, ch, D), jnp.float32),
        ] + [pltpu.SemaphoreType.DMA] * (2 * nbuf),
    )
    def gather(table_hbm, idx_hbm, out_hbm, idx_v, rows_v, *sems):
        gsem, wsem = sems[:nbuf], sems[nbuf:]
        wid = lax.axis_index("s") * info.num_cores + lax.axis_index("c")
        base = wid * b_per_w
        pltpu.sync_copy(idx_hbm.at[pl.ds(base, b_per_w)], idx_v)
        for t in range(b_per_w // 16):
            sl = pl.ds(t * 16, 16)
            idx_v[sl] = jnp.clip(idx_v[sl], 0, vocab - 1)

        def start_gather(i):
            return pltpu.async_copy(
                table_hbm.at[idx_v.at[pl.ds(i * ch, ch)]],
                rows_v.at[i % nbuf], gsem[i % nbuf])

        def start_write(i):
            return pltpu.async_copy(
                rows_v.at[i % nbuf], out_hbm.at[pl.ds(base + i * ch, ch)],
                wsem[i % nbuf])

        gathers = [None] * n_ch
        writes = [None] * n_ch
        for j in range(min(nbuf - 1, n_ch)):
            gathers[j] = start_gather(j)
        for i in range(n_ch):
            j = i + nbuf - 1
            if j < n_ch:
                if j - nbuf >= 0:
                    writes[j - nbuf].wait()
                gathers[j] = start_gather(j)
            gathers[i].wait()
            writes[i] = start_write(i)
        for i in range(max(0, n_ch - nbuf), n_ch):
            writes[i].wait()

    return gather


def kernel(ids, ids_valid, ids_mask, embed_table):
    vocab, d = embed_table.shape
    b, s = ids.shape
    ids_flat = ids.reshape(-1)
    out = _make_gather(b * s, d, vocab)(embed_table, ids_flat)
    return (out.reshape(b, s, d), ids_valid, ids, ids_mask)


# speedup vs baseline: 1.0196x; 1.0011x over previous
import functools

import jax
import jax.numpy as jnp
from jax import lax
from jax.experimental import pallas as pl
from jax.experimental.pallas import tpu as pltpu
from jax.experimental.pallas import tpu_sc as plsc


def _make_gather(Bb: int, S: int, D: int, vocab: int):
    info = plsc.get_sparse_core_info()
    nw = info.num_cores * info.num_subcores
    b_per_w = (Bb * S) // nw
    assert S % b_per_w == 0
    wpr = S // b_per_w
    ch = 16
    nbuf = 6
    n_ch = b_per_w // ch
    mesh = plsc.VectorSubcoreMesh(core_axis_name="c", subcore_axis_name="s")

    @functools.partial(
        pl.kernel,
        mesh=mesh,
        out_type=jax.ShapeDtypeStruct((Bb, S, D), jnp.float32),
        scratch_types=[
            pltpu.VMEM((b_per_w,), jnp.int32),
            pltpu.VMEM((nbuf, ch, D), jnp.float32),
        ] + [pltpu.SemaphoreType.DMA] * (2 * nbuf),
    )
    def gather(table_hbm, idx_hbm, out_hbm, idx_v, rows_v, *sems):
        gsem, wsem = sems[:nbuf], sems[nbuf:]
        wid = lax.axis_index("s") * info.num_cores + lax.axis_index("c")
        row = wid // wpr
        col = (wid % wpr) * b_per_w
        pltpu.sync_copy(idx_hbm.at[row, pl.ds(col, b_per_w)], idx_v)
        for t in range(b_per_w // 16):
            sl = pl.ds(t * 16, 16)
            idx_v[sl] = jnp.clip(idx_v[sl], 0, vocab - 1)

        def start_gather(i):
            return pltpu.async_copy(
                table_hbm.at[idx_v.at[pl.ds(i * ch, ch)]],
                rows_v.at[i % nbuf], gsem[i % nbuf])

        def start_write(i):
            return pltpu.async_copy(
                rows_v.at[i % nbuf],
                out_hbm.at[row, pl.ds(col + i * ch, ch)],
                wsem[i % nbuf])

        gathers = [None] * n_ch
        writes = [None] * n_ch
        for j in range(min(nbuf - 1, n_ch)):
            gathers[j] = start_gather(j)
        for i in range(n_ch):
            j = i + nbuf - 1
            if j < n_ch:
                if j - nbuf >= 0:
                    writes[j - nbuf].wait()
                gathers[j] = start_gather(j)
            gathers[i].wait()
            writes[i] = start_write(i)
        for i in range(max(0, n_ch - nbuf), n_ch):
            writes[i].wait()

    return gather


def kernel(ids, ids_valid, ids_mask, embed_table):
    vocab, d = embed_table.shape
    b, s = ids.shape
    out = _make_gather(b, s, d, vocab)(embed_table, ids)
    return (out, ids_valid, ids, ids_mask)
